# baseline (speedup 1.0000x reference)
import functools

import jax
import jax.numpy as jnp
from jax import lax
from jax.experimental import pallas as pl
from jax.experimental.pallas import tpu as pltpu
from jax.experimental.pallas import tpu_sc as plsc

B = 16384
D = 64
NC = 2
NS = 16
NW = NC * NS
BPW = B // NW
CH = 128
NCH = BPW // CH
L = 16
EPS = 1e-12

_mesh = plsc.VectorSubcoreMesh(core_axis_name="c", subcore_axis_name="s")

_TAKE_DNUMS = lax.GatherDimensionNumbers(
    offset_dims=(), collapsed_slice_dims=(0,), start_index_map=(0,))


def _lane_shuffle(x, perm):
    return lax.gather(x, perm[:, None], _TAKE_DNUMS, (1,),
                      mode=lax.GatherScatterMode.PROMISE_IN_BOUNDS)


@functools.partial(
    pl.kernel,
    mesh=_mesh,
    compiler_params=pltpu.CompilerParams(use_tc_tiling_on_sc=False),
    out_type=(
        jax.ShapeDtypeStruct((B, D), jnp.float32),
        jax.ShapeDtypeStruct((B, D), jnp.float32),
    ),
    scratch_types=[
        pltpu.VMEM((NCH, CH), jnp.int32),
        pltpu.VMEM((BPW, D), jnp.float32),
        pltpu.VMEM((BPW, D), jnp.float32),
        pltpu.SemaphoreType.DMA,
        pltpu.SemaphoreType.DMA,
    ],
)
def _gather_normalize(idx_hbm, tr_hbm, ti_hbm, or_hbm, oi_hbm,
                      idx_v, rr_v, ri_v, sem_r, sem_i):
    wid = lax.axis_index("s") * NC + lax.axis_index("c")
    base = wid * BPW

    pltpu.sync_copy(idx_hbm.at[wid], idx_v)

    copies = []
    for j in range(NCH):
        copies.append(pltpu.async_copy(
            tr_hbm.at[idx_v.at[j]], rr_v.at[pl.ds(j * CH, CH)], sem_r))
        copies.append(pltpu.async_copy(
            ti_hbm.at[idx_v.at[j]], ri_v.at[pl.ds(j * CH, CH)], sem_i))
    for c in copies:
        c.wait()

    def row_body(i, carry):
        r = [rr_v[i, pl.ds(L * j, L)] for j in range(D // L)]
        m = [ri_v[i, pl.ds(L * j, L)] for j in range(D // L)]
        acc = r[0] * r[0]
        for v in r[1:]:
            acc = acc + v * v
        for v in m:
            acc = acc + v * v
        lanes = lax.iota(jnp.int32, L)
        for sh in (8, 4, 2, 1):
            perm = lanes ^ jnp.int32(sh)
            acc = acc + _lane_shuffle(acc, perm)
        sv = acc + jnp.float32(EPS)
        bits = lax.bitcast_convert_type(sv, jnp.int32)
        bits = jnp.int32(0x5F3759DF) - (bits >> 1)
        y = lax.bitcast_convert_type(bits, jnp.float32)
        for _ in range(3):
            y = y * (jnp.float32(1.5) - jnp.float32(0.5) * sv * y * y)
        for j in range(D // L):
            rr_v[i, pl.ds(L * j, L)] = r[j] * y
            ri_v[i, pl.ds(L * j, L)] = m[j] * y
        return carry

    lax.fori_loop(0, BPW, row_body, 0)

    pltpu.sync_copy(rr_v, or_hbm.at[pl.ds(base, BPW)])
    pltpu.sync_copy(ri_v, oi_hbm.at[pl.ds(base, BPW)])


def kernel(subcarrier_indices, basis_real, basis_imag):
    idx = subcarrier_indices.astype(jnp.int32).reshape(NW, NCH, CH)
    out_r, out_i = _gather_normalize(idx, basis_real, basis_imag)
    return lax.complex(out_r, out_i)


# speedup vs baseline: 1.0045x; 1.0045x over previous
import functools

import jax
import jax.numpy as jnp
from jax import lax
from jax.experimental import pallas as pl
from jax.experimental.pallas import tpu as pltpu
from jax.experimental.pallas import tpu_sc as plsc

B = 16384
D = 64
NC = 2
NS = 16
NW = NC * NS
BPW = B // NW
CH = 128
NCH = BPW // CH
EPS = 1e-12
TBLK = 1024

_mesh = plsc.VectorSubcoreMesh(core_axis_name="c", subcore_axis_name="s")


@functools.partial(
    pl.kernel,
    mesh=_mesh,
    compiler_params=pltpu.CompilerParams(use_tc_tiling_on_sc=False),
    out_type=(
        jax.ShapeDtypeStruct((B, D), jnp.float32),
        jax.ShapeDtypeStruct((B, D), jnp.float32),
    ),
    scratch_types=[
        pltpu.VMEM((NCH, CH), jnp.int32),
        pltpu.VMEM((BPW, D), jnp.float32),
        pltpu.VMEM((BPW, D), jnp.float32),
        pltpu.SemaphoreType.DMA,
        pltpu.SemaphoreType.DMA,
    ],
)
def _gather(idx_hbm, tr_hbm, ti_hbm, or_hbm, oi_hbm,
            idx_v, rr_v, ri_v, sem_r, sem_i):
    wid = lax.axis_index("s") * NC + lax.axis_index("c")
    base = wid * BPW

    pltpu.sync_copy(idx_hbm.at[wid], idx_v)

    copies = []
    for j in range(NCH):
        copies.append(pltpu.async_copy(
            tr_hbm.at[idx_v.at[j]], rr_v.at[pl.ds(j * CH, CH)], sem_r))
        copies.append(pltpu.async_copy(
            ti_hbm.at[idx_v.at[j]], ri_v.at[pl.ds(j * CH, CH)], sem_i))
    for c in copies:
        c.wait()

    pltpu.sync_copy(rr_v, or_hbm.at[pl.ds(base, BPW)])
    pltpu.sync_copy(ri_v, oi_hbm.at[pl.ds(base, BPW)])


def _normalize_body(rr_ref, ri_ref, or_ref, oi_ref):
    r = rr_ref[...]
    m = ri_ref[...]
    s = jnp.sum(r * r + m * m, axis=1, keepdims=True) + jnp.float32(EPS)
    inv = lax.rsqrt(s)
    or_ref[...] = r * inv
    oi_ref[...] = m * inv


_normalize = pl.pallas_call(
    _normalize_body,
    grid=(B // TBLK,),
    in_specs=[pl.BlockSpec((TBLK, D), lambda i: (i, 0)),
              pl.BlockSpec((TBLK, D), lambda i: (i, 0))],
    out_specs=[pl.BlockSpec((TBLK, D), lambda i: (i, 0)),
               pl.BlockSpec((TBLK, D), lambda i: (i, 0))],
    out_shape=[jax.ShapeDtypeStruct((B, D), jnp.float32),
               jax.ShapeDtypeStruct((B, D), jnp.float32)],
)


def kernel(subcarrier_indices, basis_real, basis_imag):
    idx = subcarrier_indices.astype(jnp.int32).reshape(NW, NCH, CH)
    g_r, g_i = _gather(idx, basis_real, basis_imag)
    n_r, n_i = _normalize(g_r, g_i)
    return lax.complex(n_r, n_i)


# speedup vs baseline: 1.1233x; 1.1182x over previous
import functools

import jax
import jax.numpy as jnp
from jax import lax
from jax.experimental import pallas as pl
from jax.experimental.pallas import tpu as pltpu
from jax.experimental.pallas import tpu_sc as plsc

B = 16384
D = 64
W = 2 * D
NC = 2
NS = 16
NW = NC * NS
BPW = B // NW
CH = 128
NCH = BPW // CH
EPS = 1e-12
TBLK = 1024

_mesh = plsc.VectorSubcoreMesh(core_axis_name="c", subcore_axis_name="s")


@functools.partial(
    pl.kernel,
    mesh=_mesh,
    compiler_params=pltpu.CompilerParams(use_tc_tiling_on_sc=True),
    out_type=jax.ShapeDtypeStruct((B, W), jnp.float32),
    scratch_types=[
        pltpu.VMEM((BPW,), jnp.int32),
        pltpu.VMEM((BPW, W), jnp.float32),
        pltpu.SemaphoreType.DMA,
    ],
)
def _gather(idx_hbm, tab_hbm, out_hbm, idx_v, rows_v, sem):
    wid = lax.axis_index("s") * NC + lax.axis_index("c")
    base = wid * BPW

    pltpu.sync_copy(idx_hbm.at[pl.ds(base, BPW)], idx_v)

    copies = []
    for j in range(NCH):
        copies.append(pltpu.async_copy(
            tab_hbm.at[idx_v.at[pl.ds(j * CH, CH)]],
            rows_v.at[pl.ds(j * CH, CH)], sem))
    for c in copies:
        c.wait()

    pltpu.sync_copy(rows_v, out_hbm.at[pl.ds(base, BPW)])


def _normalize_body(x_ref, or_ref, oi_ref):
    x = x_ref[...]
    r = x[:, :D]
    m = x[:, D:]
    s = jnp.sum(r * r + m * m, axis=1, keepdims=True) + jnp.float32(EPS)
    inv = lax.rsqrt(s)
    or_ref[...] = r * inv
    oi_ref[...] = m * inv


_normalize = pl.pallas_call(
    _normalize_body,
    grid=(B // TBLK,),
    in_specs=[pl.BlockSpec((TBLK, W), lambda i: (i, 0))],
    out_specs=[pl.BlockSpec((TBLK, D), lambda i: (i, 0)),
               pl.BlockSpec((TBLK, D), lambda i: (i, 0))],
    out_shape=[jax.ShapeDtypeStruct((B, D), jnp.float32),
               jax.ShapeDtypeStruct((B, D), jnp.float32)],
)


def kernel(subcarrier_indices, basis_real, basis_imag):
    idx = subcarrier_indices.astype(jnp.int32)
    tab = jnp.concatenate([basis_real, basis_imag], axis=1)
    g = _gather(idx, tab)
    n_r, n_i = _normalize(g)
    return lax.complex(n_r, n_i)
